# Initial kernel scaffold; baseline (speedup 1.0000x reference)
import jax
import jax.numpy as jnp
from jax.experimental import pallas as pl


def kernel(x, edge_index, W1, b1, W2, b2):
    raise NotImplementedError("write your pallas kernel here")


# speedup vs baseline: 2.1944x; 2.1944x over previous
import functools

import jax
import jax.numpy as jnp
from jax import lax
from jax.experimental import pallas as pl
from jax.experimental.pallas import tpu as pltpu
from jax.experimental.pallas import tpu_sc as plsc

N = 10000
E = 320000
D = 128

_NC = 2
_NS = 16
_NW = _NC * _NS
_EPW = E // _NW
_C = 80
_NCH = _EPW // _C


def _pre_body(x_ref, w1_ref, b1_ref, p_ref, q_ref):
    xv = x_ref[...]
    wa = w1_ref[:D, :] - w1_ref[D:, :]
    wb = w1_ref[D:, :]
    p_ref[...] = jnp.dot(xv, wa, preferred_element_type=jnp.float32) + b1_ref[...]
    q_ref[...] = jnp.dot(xv, wb, preferred_element_type=jnp.float32)


def _precompute(x, W1, b1):
    return pl.pallas_call(
        _pre_body,
        out_shape=(
            jax.ShapeDtypeStruct((N, D), jnp.float32),
            jax.ShapeDtypeStruct((N, D), jnp.float32),
        ),
    )(x, W1, b1.reshape(1, D))


def _edge_body(p_hbm, q_hbm, row_hbm, col_hbm, out_hbm,
               idxr_v, idxc_v, prow_v, qrow_v, semp, semq):
    wid = lax.axis_index("s") * _NC + lax.axis_index("c")
    base = wid * _EPW

    pltpu.sync_copy(row_hbm.at[wid], idxr_v)
    pltpu.sync_copy(col_hbm.at[wid], idxc_v)

    def chunk(i, carry):
        gp = pltpu.async_copy(p_hbm.at[idxr_v.at[i]], prow_v, semp)
        gq = pltpu.async_copy(q_hbm.at[idxc_v.at[i]], qrow_v, semq)
        gp.wait()
        gq.wait()

        def rowfn(r, c2):
            for j in range(D // 16):
                sl = pl.ds(j * 16, 16)
                prow_v[r, sl] = jnp.maximum(prow_v[r, sl] + qrow_v[r, sl], 0.0)
            return c2

        lax.fori_loop(0, _C, rowfn, 0, unroll=2)
        pltpu.sync_copy(prow_v, out_hbm.at[pl.ds(base + i * _C, _C)])
        return carry

    lax.fori_loop(0, _NCH, chunk, 0)


def _edge_stage(P, Q, row, col):
    mesh = plsc.VectorSubcoreMesh(core_axis_name="c", subcore_axis_name="s")
    fn = pl.kernel(
        _edge_body,
        out_type=jax.ShapeDtypeStruct((E, D), jnp.float32),
        mesh=mesh,
        scratch_types=[
            pltpu.VMEM((_NCH, _C), jnp.int32),
            pltpu.VMEM((_NCH, _C), jnp.int32),
            pltpu.VMEM((_C, D), jnp.float32),
            pltpu.VMEM((_C, D), jnp.float32),
            pltpu.SemaphoreType.DMA,
            pltpu.SemaphoreType.DMA,
        ],
    )
    return fn(P, Q, row.reshape(_NW, _NCH, _C), col.reshape(_NW, _NCH, _C))


_BE = 4000


def _mlp_body(h_ref, w2_ref, b2_ref, o_ref):
    o_ref[...] = jnp.maximum(
        jnp.dot(h_ref[...], w2_ref[...], preferred_element_type=jnp.float32)
        + b2_ref[...],
        0.0,
    )


def _mlp2(h, W2, b2):
    grid = (E // _BE,)
    return pl.pallas_call(
        _mlp_body,
        grid=grid,
        in_specs=[
            pl.BlockSpec((_BE, D), lambda i: (i, 0)),
            pl.BlockSpec((D, D), lambda i: (0, 0)),
            pl.BlockSpec((1, D), lambda i: (0, 0)),
        ],
        out_specs=pl.BlockSpec((_BE, D), lambda i: (i, 0)),
        out_shape=jax.ShapeDtypeStruct((E, D), jnp.float32),
    )(h, W2, b2.reshape(1, D))


@jax.jit
def kernel(x, edge_index, W1, b1, W2, b2):
    row = edge_index[0].astype(jnp.int32)
    col = edge_index[1].astype(jnp.int32)
    P, Q = _precompute(x, W1, b1)
    h = _edge_stage(P, Q, row, col)
    return _mlp2(h, W2, b2)


# speedup vs baseline: 2.8330x; 1.2910x over previous
import functools

import jax
import jax.numpy as jnp
from jax import lax
from jax.experimental import pallas as pl
from jax.experimental.pallas import tpu as pltpu
from jax.experimental.pallas import tpu_sc as plsc

N = 10000
E = 320000
D = 128

_NC = 2
_NS = 16
_NW = _NC * _NS
_EPW = E // _NW
_C = 80
_NCH = _EPW // _C


def _pre_body(x_ref, w1_ref, b1_ref, p_ref, q_ref):
    xv = x_ref[...]
    wa = w1_ref[:D, :] - w1_ref[D:, :]
    wb = w1_ref[D:, :]
    p_ref[...] = jnp.dot(xv, wa, preferred_element_type=jnp.float32) + b1_ref[...]
    q_ref[...] = jnp.dot(xv, wb, preferred_element_type=jnp.float32)


def _precompute(x, W1, b1):
    return pl.pallas_call(
        _pre_body,
        out_shape=(
            jax.ShapeDtypeStruct((N, D), jnp.float32),
            jax.ShapeDtypeStruct((N, D), jnp.float32),
        ),
    )(x, W1, b1.reshape(1, D))


def _edge_body(p_hbm, q_hbm, row_hbm, col_hbm, out_hbm,
               idxr_v, idxc_v, prow_v, qrow_v, h_v, semp, semq, semo):
    wid = lax.axis_index("s") * _NC + lax.axis_index("c")
    base = wid * _EPW

    pltpu.sync_copy(row_hbm.at[wid], idxr_v)
    pltpu.sync_copy(col_hbm.at[wid], idxc_v)

    def issue_gather(i, b):
        pltpu.async_copy(p_hbm.at[idxr_v.at[i]], prow_v[b], semp[b])
        pltpu.async_copy(q_hbm.at[idxc_v.at[i]], qrow_v[b], semq[b])

    def wait_gather(b):
        pltpu.make_async_copy(p_hbm.at[idxr_v.at[0]], prow_v[b], semp[b]).wait()
        pltpu.make_async_copy(q_hbm.at[idxc_v.at[0]], qrow_v[b], semq[b]).wait()

    def wait_store(b):
        pltpu.make_async_copy(h_v[b], out_hbm.at[pl.ds(base, _C)], semo[b]).wait()

    def compute(b):
        def rowfn(r, c2):
            for j in range(D // 16):
                sl = pl.ds(j * 16, 16)
                h_v[b][r, sl] = jnp.maximum(prow_v[b][r, sl] + qrow_v[b][r, sl], 0.0)
            return c2

        lax.fori_loop(0, _C, rowfn, 0, unroll=2)

    issue_gather(0, 0)
    issue_gather(1, 1)

    def pair(g, carry):
        for b in range(2):
            i = 2 * g + b
            wait_gather(b)
            pl.when(g > 0)(lambda: wait_store(b))
            compute(b)
            pl.when(i + 2 < _NCH)(lambda: issue_gather(i + 2, b))
            pltpu.async_copy(h_v[b], out_hbm.at[pl.ds(base + i * _C, _C)], semo[b])
        return carry

    lax.fori_loop(0, _NCH // 2, pair, 0)

    i_last = _NCH - 1
    wait_gather(0)
    wait_store(0)
    compute(0)
    pltpu.async_copy(h_v[0], out_hbm.at[pl.ds(base + i_last * _C, _C)], semo[0])
    wait_store(0)
    wait_store(1)


def _edge_stage(P, Q, row, col):
    mesh = plsc.VectorSubcoreMesh(core_axis_name="c", subcore_axis_name="s")
    fn = pl.kernel(
        _edge_body,
        out_type=jax.ShapeDtypeStruct((E, D), jnp.float32),
        mesh=mesh,
        scratch_types=[
            pltpu.VMEM((_NCH, _C), jnp.int32),
            pltpu.VMEM((_NCH, _C), jnp.int32),
            [pltpu.VMEM((_C, D), jnp.float32)] * 2,
            [pltpu.VMEM((_C, D), jnp.float32)] * 2,
            [pltpu.VMEM((_C, D), jnp.float32)] * 2,
            [pltpu.SemaphoreType.DMA] * 2,
            [pltpu.SemaphoreType.DMA] * 2,
            [pltpu.SemaphoreType.DMA] * 2,
        ],
    )
    return fn(P, Q, row.reshape(_NW, _NCH, _C), col.reshape(_NW, _NCH, _C))


_BE = 4000


def _mlp_body(h_ref, w2_ref, b2_ref, o_ref):
    o_ref[...] = jnp.maximum(
        jnp.dot(h_ref[...], w2_ref[...], preferred_element_type=jnp.float32)
        + b2_ref[...],
        0.0,
    )


def _mlp2(h, W2, b2):
    grid = (E // _BE,)
    return pl.pallas_call(
        _mlp_body,
        grid=grid,
        in_specs=[
            pl.BlockSpec((_BE, D), lambda i: (i, 0)),
            pl.BlockSpec((D, D), lambda i: (0, 0)),
            pl.BlockSpec((1, D), lambda i: (0, 0)),
        ],
        out_specs=pl.BlockSpec((_BE, D), lambda i: (i, 0)),
        out_shape=jax.ShapeDtypeStruct((E, D), jnp.float32),
    )(h, W2, b2.reshape(1, D))


@jax.jit
def kernel(x, edge_index, W1, b1, W2, b2):
    row = edge_index[0].astype(jnp.int32)
    col = edge_index[1].astype(jnp.int32)
    P, Q = _precompute(x, W1, b1)
    h = _edge_stage(P, Q, row, col)
    return _mlp2(h, W2, b2)


# speedup vs baseline: 2.8335x; 1.0002x over previous
import functools

import jax
import jax.numpy as jnp
from jax import lax
from jax.experimental import pallas as pl
from jax.experimental.pallas import tpu as pltpu
from jax.experimental.pallas import tpu_sc as plsc

N = 10000
E = 320000
D = 128

_NC = 2
_NS = 16
_NW = _NC * _NS
_EPW = E // _NW
_C = 80
_NCH = _EPW // _C
_NB = 3


def _pre_body(x_ref, w1_ref, b1_ref, p_ref, q_ref):
    xv = x_ref[...]
    wa = w1_ref[:D, :] - w1_ref[D:, :]
    wb = w1_ref[D:, :]
    p_ref[...] = jnp.dot(xv, wa, preferred_element_type=jnp.float32) + b1_ref[...]
    q_ref[...] = jnp.dot(xv, wb, preferred_element_type=jnp.float32)


def _precompute(x, W1, b1):
    return pl.pallas_call(
        _pre_body,
        out_shape=(
            jax.ShapeDtypeStruct((N, D), jnp.float32),
            jax.ShapeDtypeStruct((N, D), jnp.float32),
        ),
    )(x, W1, b1.reshape(1, D))


def _edge_body(p_hbm, q_hbm, row_hbm, col_hbm, out_hbm,
               idxr_v, idxc_v, prow_v, qrow_v, h_v, semp, semq, semo):
    wid = lax.axis_index("s") * _NC + lax.axis_index("c")
    base = wid * _EPW

    pltpu.sync_copy(row_hbm.at[wid], idxr_v)
    pltpu.sync_copy(col_hbm.at[wid], idxc_v)

    def issue_gather(i, b):
        pltpu.async_copy(p_hbm.at[idxr_v.at[i]], prow_v[b], semp[b])
        pltpu.async_copy(q_hbm.at[idxc_v.at[i]], qrow_v[b], semq[b])

    def wait_gather(b):
        pltpu.make_async_copy(p_hbm.at[idxr_v.at[0]], prow_v[b], semp[b]).wait()
        pltpu.make_async_copy(q_hbm.at[idxc_v.at[0]], qrow_v[b], semq[b]).wait()

    def wait_store(b):
        pltpu.make_async_copy(h_v[b], out_hbm.at[pl.ds(base, _C)], semo[b]).wait()

    def compute(b):
        def rowfn(r, c2):
            for j in range(D // 16):
                sl = pl.ds(j * 16, 16)
                h_v[b][r, sl] = jnp.maximum(prow_v[b][r, sl] + qrow_v[b][r, sl],
                                            0.0)
            return c2

        lax.fori_loop(0, _C, rowfn, 0, unroll=4)

    for b in range(_NB):
        issue_gather(b, b)

    def group(g, carry):
        for b in range(_NB):
            i = _NB * g + b
            wait_gather(b)
            pl.when(g > 0)(lambda: wait_store(b))
            compute(b)
            pl.when(i + _NB < _NCH)(lambda: issue_gather(i + _NB, b))
            pltpu.async_copy(h_v[b], out_hbm.at[pl.ds(base + i * _C, _C)],
                             semo[b])
        return carry

    lax.fori_loop(0, _NCH // _NB, group, 0)

    for k in range(_NCH - (_NCH // _NB) * _NB):
        i = (_NCH // _NB) * _NB + k
        wait_gather(k)
        wait_store(k)
        compute(k)
        pltpu.async_copy(h_v[k], out_hbm.at[pl.ds(base + i * _C, _C)], semo[k])

    for b in range(_NB):
        wait_store(b)


def _edge_stage(P, Q, row, col):
    mesh = plsc.VectorSubcoreMesh(core_axis_name="c", subcore_axis_name="s")
    fn = pl.kernel(
        _edge_body,
        out_type=jax.ShapeDtypeStruct((E, D), jnp.float32),
        mesh=mesh,
        scratch_types=[
            pltpu.VMEM((_NCH, _C), jnp.int32),
            pltpu.VMEM((_NCH, _C), jnp.int32),
            [pltpu.VMEM((_C, D), jnp.float32)] * _NB,
            [pltpu.VMEM((_C, D), jnp.float32)] * _NB,
            [pltpu.VMEM((_C, D), jnp.float32)] * _NB,
            [pltpu.SemaphoreType.DMA] * _NB,
            [pltpu.SemaphoreType.DMA] * _NB,
            [pltpu.SemaphoreType.DMA] * _NB,
        ],
    )
    return fn(P, Q, row.reshape(_NW, _NCH, _C), col.reshape(_NW, _NCH, _C))


_BE = 4000


def _mlp_body(h_ref, w2_ref, b2_ref, o_ref):
    o_ref[...] = jnp.maximum(
        jnp.dot(h_ref[...], w2_ref[...], preferred_element_type=jnp.float32)
        + b2_ref[...],
        0.0,
    )


def _mlp2(h, W2, b2):
    grid = (E // _BE,)
    return pl.pallas_call(
        _mlp_body,
        grid=grid,
        in_specs=[
            pl.BlockSpec((_BE, D), lambda i: (i, 0)),
            pl.BlockSpec((D, D), lambda i: (0, 0)),
            pl.BlockSpec((1, D), lambda i: (0, 0)),
        ],
        out_specs=pl.BlockSpec((_BE, D), lambda i: (i, 0)),
        out_shape=jax.ShapeDtypeStruct((E, D), jnp.float32),
    )(h, W2, b2.reshape(1, D))


@jax.jit
def kernel(x, edge_index, W1, b1, W2, b2):
    row = edge_index[0].astype(jnp.int32)
    col = edge_index[1].astype(jnp.int32)
    P, Q = _precompute(x, W1, b1)
    h = _edge_stage(P, Q, row, col)
    return _mlp2(h, W2, b2)
